# baseline (speedup 1.0000x reference)
import functools

import jax
import jax.numpy as jnp
from jax import lax
from jax.experimental import pallas as pl
from jax.experimental.pallas import tpu as pltpu
from jax.experimental.pallas import tpu_sc as plsc

N = 10000
E = 320000
RK = 3
SK = 3
H = 32
NC = 2
NS = 16
NW = NC * NS
EPW = E // NW
C = 80
NCH = EPW // C
GCH = 25
NGR = NCH // GCH
NP = 10240
TPT = NP // NS
TABP = 2 * NP
TPW = TABP // NS

E4 = E // 4
C4 = C // 4
EB4 = 2000
NBE = E4 // EB4
N4 = N // 4
NP4 = NP // 4
NB4 = N4
NBN = N4 // NB4
THOP = N // C

_mesh = plsc.VectorSubcoreMesh(core_axis_name="c", subcore_axis_name="s")


@functools.partial(
    pl.kernel,
    out_type=jax.ShapeDtypeStruct((E4, 4 * H), jnp.float32),
    mesh=_mesh,
    scratch_types=[
        pltpu.VMEM((6, GCH, C), jnp.int32),
        pltpu.VMEM((6, C, H), jnp.float32),
        pltpu.VMEM((C4, 4 * H), jnp.float32),
        pltpu.VMEM((C, H), jnp.float32),
        pltpu.VMEM_SHARED((TABP, H), jnp.float32),
        pltpu.SemaphoreType.DMA,
        pltpu.SemaphoreType.DMA,
    ],
    compiler_params=pltpu.CompilerParams(use_tc_tiling_on_sc=False),
)
def _sc_gather(trp_hbm, tsp_hbm, rs_hbm, out_hbm, idx6, rb, ob, stg, tab_sh,
               gsem, osem):
    c = lax.axis_index("c")
    s = lax.axis_index("s")
    wid = c * NS + s

    def stage_hop(it, carry):
        hh = s + it * NS

        @pl.when(hh < THOP)
        def _():
            for t, tp in enumerate([trp_hbm, tsp_hbm]):
                pltpu.sync_copy(tp.at[pl.ds(hh * C4, C4)], ob)

                def unpk(pr, carry2):
                    for slot in range(4):
                        for cc in range(2):
                            stg[4 * pr + slot, pl.ds(cc * 16, 16)] = (
                                ob[pr, pl.ds(slot * 32 + cc * 16, 16)])
                    return carry2

                lax.fori_loop(0, C4, unpk, 0)
                pltpu.sync_copy(stg, tab_sh.at[pl.ds(t * NP + hh * C, C)])
        return carry

    lax.fori_loop(0, (THOP + NS - 1) // NS, stage_hop, 0)
    plsc.subcore_barrier()

    def group(g, carry):
        for k in range(6):
            pltpu.sync_copy(rs_hbm.at[(k * NW + wid) * NGR + g], idx6.at[k])

        def chunk(j, carry2):
            descs = []
            for k in range(6):
                descs.append(
                    pltpu.async_copy(tab_sh.at[idx6.at[k, j]], rb.at[k], gsem))
            for d in descs:
                d.wait()

            def rowloop(pr, carry3):
                for slot in range(4):
                    r = 4 * pr + slot
                    for cc in range(2):
                        sl = pl.ds(cc * 16, 16)
                        osl = pl.ds(slot * 32 + cc * 16, 16)
                        ob[pr, osl] = (
                            rb[0, r, sl] + rb[1, r, sl] + rb[2, r, sl]
                            + rb[3, r, sl] + rb[4, r, sl] + rb[5, r, sl])
                return carry3

            lax.fori_loop(0, C4, rowloop, 0)
            pltpu.async_copy(
                ob,
                out_hbm.at[pl.ds((wid * EPW + (g * GCH + j) * C) // 4, C4)],
                osem).wait()
            return carry2

        lax.fori_loop(0, GCH, chunk, 0)
        return carry

    lax.fori_loop(0, NGR, group, 0)


def _make_ident(rows):
    @functools.partial(
        pl.kernel,
        out_type=jax.ShapeDtypeStruct((rows, GCH, C), jnp.int32),
        mesh=_mesh,
        scratch_types=[
            pltpu.VMEM((GCH, C), jnp.int32),
        ],
        compiler_params=pltpu.CompilerParams(use_tc_tiling_on_sc=False),
    )
    def _ident(src_hbm, out_hbm, buf):
        c = lax.axis_index("c")
        s = lax.axis_index("s")
        wid = c * NS + s
        per = rows // NW

        def body(i, carry):
            pltpu.sync_copy(src_hbm.at[wid * per + i], buf)
            pltpu.sync_copy(buf, out_hbm.at[wid * per + i])
            return carry

        lax.fori_loop(0, per, body, 0)

    return _ident


_ident_rs = _make_ident(6 * NW * NGR)
_ident_r3 = _make_ident(3 * NW * NGR)


@functools.partial(
    pl.kernel,
    out_type=jax.ShapeDtypeStruct((NC * NP4, 4 * H), jnp.float32),
    mesh=_mesh,
    scratch_types=[
        pltpu.VMEM((3, GCH, C), jnp.int32),
        pltpu.VMEM((C, H), jnp.float32),
        pltpu.VMEM((C4, 4 * H), jnp.float32),
        pltpu.VMEM_SHARED((NP, H), jnp.float32),
    ],
    compiler_params=pltpu.CompilerParams(use_tc_tiling_on_sc=False),
)
def _sc_scatter(he_hbm, r_hbm, out_hbm, idx3, heb, heb2, acc_sh):
    c = lax.axis_index("c")
    s = lax.axis_index("s")
    wid = c * NS + s

    def zrow(i, carry):
        heb[i, pl.ds(0, 16)] = jnp.zeros((16,), jnp.float32)
        heb[i, pl.ds(16, 16)] = jnp.zeros((16,), jnp.float32)
        return carry

    lax.fori_loop(0, C, zrow, 0)

    def zhop(h, carry):
        pltpu.sync_copy(heb, acc_sh.at[pl.ds(s * TPT + h * C, C)])
        return carry

    lax.fori_loop(0, TPT // C, zhop, 0)
    plsc.subcore_barrier()

    def group(g, carry):
        for k in range(3):
            pltpu.sync_copy(r_hbm.at[(k * NW + wid) * NGR + g], idx3.at[k])

        def chunk(j, carry2):
            pltpu.sync_copy(
                he_hbm.at[pl.ds((wid * EPW + (g * GCH + j) * C) // 4, C4)],
                heb2)

            def unpack(pr, carry3):
                for slot in range(4):
                    for cc in range(2):
                        heb[4 * pr + slot, pl.ds(cc * 16, 16)] = (
                            heb2[pr, pl.ds(slot * 32 + cc * 16, 16)])
                return carry3

            lax.fori_loop(0, C4, unpack, 0)
            for k in range(3):
                pltpu.sync_copy(heb, acc_sh.at[idx3.at[k, j]], add=True)
            return carry2

        lax.fori_loop(0, GCH, chunk, 0)
        return carry

    lax.fori_loop(0, NGR, group, 0)
    plsc.subcore_barrier()

    def ohop(h, carry):
        row0 = s * TPT + h * C
        pltpu.sync_copy(acc_sh.at[pl.ds(row0, C)], heb)

        def pk(pr, carry2):
            for slot in range(4):
                for cc in range(2):
                    heb2[pr, pl.ds(slot * 32 + cc * 16, 16)] = (
                        heb[4 * pr + slot, pl.ds(cc * 16, 16)])
            return carry2

        lax.fori_loop(0, C4, pk, 0)
        pltpu.sync_copy(heb2, out_hbm.at[pl.ds((c * NP + row0) // 4, C4)])
        return carry

    lax.fori_loop(0, TPT // C, ohop, 0)


def _enc_nodes_body(x_ref, w_ref, w2_ref, w3_ref, hn_ref, tr_ref, ts_ref):
    h = jnp.maximum(
        jnp.dot(x_ref[...], w_ref[...], preferred_element_type=jnp.float32), 0.0)
    hn_ref[...] = h
    tr_ref[...] = jnp.dot(h, w2_ref[...], preferred_element_type=jnp.float32)
    ts_ref[...] = jnp.dot(h, w3_ref[...], preferred_element_type=jnp.float32)


def _enc_edges_body(x_ref, w_ref, he_ref):
    he_ref[...] = jnp.maximum(
        jnp.dot(x_ref[...], w_ref[...], preferred_element_type=jnp.float32), 0.0)


def _edge_body(he_ref, g_ref, n0_ref, w1_ref, gv_ref, f_ref, out_ref, agg_ref):
    b = pl.program_id(0)
    u = jnp.dot(he_ref[...], w1_ref[...],
                preferred_element_type=jnp.float32) + g_ref[...]
    ri = lax.broadcasted_iota(jnp.int32, (EB4, 128), 0) + b * EB4
    li = lax.broadcasted_iota(jnp.int32, (EB4, 128), 1)
    eidx = ri * 4 + (li >> 5)
    m0 = (eidx.astype(jnp.float32) < n0_ref[0, 0]).astype(jnp.float32)
    u = u + m0 * gv_ref[0:1, :] + (1.0 - m0) * gv_ref[1:2, :]
    hen = jnp.maximum(u, 0.0)
    out_ref[...] = hen
    s0 = jnp.sum(m0 * hen, axis=0, keepdims=True)
    st = jnp.sum(hen, axis=0, keepdims=True)
    p0 = jnp.dot(s0, f_ref[...], preferred_element_type=jnp.float32)
    pt = jnp.dot(st, f_ref[...], preferred_element_type=jnp.float32)
    part = jnp.concatenate([p0, pt - p0], axis=0)

    @pl.when(b == 0)
    def _():
        agg_ref[...] = part

    @pl.when(b != 0)
    def _():
        agg_ref[...] += part


def _node_body(hn_ref, a0_ref, a1_ref, n0_ref, w1_ref, w2_ref, we2_ref,
               we3_ref, f_ref, out_ref, tr_ref, ts_ref, agg_ref):
    b = pl.program_id(0)
    u = (jnp.dot(hn_ref[...], w1_ref[...], preferred_element_type=jnp.float32)
         + jnp.dot(a0_ref[...] + a1_ref[...], w2_ref[...],
                   preferred_element_type=jnp.float32))
    hnn = jnp.maximum(u, 0.0)
    out_ref[...] = hnn
    tr_ref[...] = jnp.dot(hnn, we2_ref[...], preferred_element_type=jnp.float32)
    ts_ref[...] = jnp.dot(hnn, we3_ref[...], preferred_element_type=jnp.float32)
    ri = lax.broadcasted_iota(jnp.int32, (NB4, 128), 0) + b * NB4
    li = lax.broadcasted_iota(jnp.int32, (NB4, 128), 1)
    nidx = ri * 4 + (li >> 5)
    m0 = (nidx.astype(jnp.float32) < n0_ref[0, 0]).astype(jnp.float32)
    s0 = jnp.sum(m0 * hnn, axis=0, keepdims=True)
    st = jnp.sum(hnn, axis=0, keepdims=True)
    p0 = jnp.dot(s0, f_ref[...], preferred_element_type=jnp.float32)
    pt = jnp.dot(st, f_ref[...], preferred_element_type=jnp.float32)
    part = jnp.concatenate([p0, pt - p0], axis=0)

    @pl.when(b == 0)
    def _():
        agg_ref[...] = part

    @pl.when(b != 0)
    def _():
        agg_ref[...] += part


def _wspec(shape):
    return pl.BlockSpec(shape, lambda i: (0,) * len(shape))


_enc_nodes = pl.pallas_call(
    _enc_nodes_body,
    grid=(NBN,),
    in_specs=[
        pl.BlockSpec((NB4, 512), lambda i: (i, 0)),
        _wspec((512, 4 * H)), _wspec((4 * H, 4 * H)), _wspec((4 * H, 4 * H)),
    ],
    out_specs=[
        pl.BlockSpec((NB4, 4 * H), lambda i: (i, 0)),
        pl.BlockSpec((NB4, 4 * H), lambda i: (i, 0)),
        pl.BlockSpec((NB4, 4 * H), lambda i: (i, 0)),
    ],
    out_shape=[jax.ShapeDtypeStruct((N4, 4 * H), jnp.float32)] * 3,
)

_enc_edges = pl.pallas_call(
    _enc_edges_body,
    grid=(NBE,),
    in_specs=[pl.BlockSpec((EB4, 64), lambda i: (i, 0)), _wspec((64, 4 * H))],
    out_specs=pl.BlockSpec((EB4, 4 * H), lambda i: (i, 0)),
    out_shape=jax.ShapeDtypeStruct((E4, 4 * H), jnp.float32),
)

_edge_step = pl.pallas_call(
    _edge_body,
    grid=(NBE,),
    in_specs=[
        pl.BlockSpec((EB4, 4 * H), lambda i: (i, 0)),
        pl.BlockSpec((EB4, 4 * H), lambda i: (i, 0)),
        _wspec((1, 1)),
        _wspec((4 * H, 4 * H)), _wspec((2, 4 * H)), _wspec((4 * H, H)),
    ],
    out_specs=[
        pl.BlockSpec((EB4, 4 * H), lambda i: (i, 0)),
        pl.BlockSpec((2, H), lambda i: (0, 0)),
    ],
    out_shape=[
        jax.ShapeDtypeStruct((E4, 4 * H), jnp.float32),
        jax.ShapeDtypeStruct((2, H), jnp.float32),
    ],
)

_node_step = pl.pallas_call(
    _node_body,
    grid=(NBN,),
    in_specs=[
        pl.BlockSpec((NB4, 4 * H), lambda i: (i, 0)),
        pl.BlockSpec((NB4, 4 * H), lambda i: (i, 0)),
        pl.BlockSpec((NB4, 4 * H), lambda i: (i, 0)),
        _wspec((1, 1)),
        _wspec((4 * H, 4 * H)), _wspec((4 * H, 4 * H)),
        _wspec((4 * H, 4 * H)), _wspec((4 * H, 4 * H)), _wspec((4 * H, H)),
    ],
    out_specs=[
        pl.BlockSpec((NB4, 4 * H), lambda i: (i, 0)),
        pl.BlockSpec((NB4, 4 * H), lambda i: (i, 0)),
        pl.BlockSpec((NB4, 4 * H), lambda i: (i, 0)),
        pl.BlockSpec((2, H), lambda i: (0, 0)),
    ],
    out_shape=[
        jax.ShapeDtypeStruct((N4, 4 * H), jnp.float32),
        jax.ShapeDtypeStruct((N4, 4 * H), jnp.float32),
        jax.ShapeDtypeStruct((N4, 4 * H), jnp.float32),
        jax.ShapeDtypeStruct((2, H), jnp.float32),
    ],
)


def kernel(nodes, edges, globals_, receivers, senders, node_graph_ids,
           edge_graph_ids, W_enc_n, W_enc_e, W_enc_g, W_e, W_n, W_g,
           W_dec, w_rank):
    recv = receivers.astype(jnp.int32)
    send = senders.astype(jnp.int32)
    rs = jnp.concatenate([recv.T, send.T + NP], axis=0)
    rs = _ident_rs(rs.reshape(6 * NW * NGR, GCH, C))
    r3 = _ident_r3(recv.T.reshape(3 * NW * NGR, GCH, C))

    We1, We2, We3, We4 = (W_e[0:H], W_e[H:2 * H], W_e[2 * H:3 * H], W_e[3 * H:])
    Wn1, Wn2 = W_n[0:H], W_n[H:]

    n0e = jnp.searchsorted(edge_graph_ids, 1)
    n0n = jnp.searchsorted(node_graph_ids, 1)
    n0ef = n0e.astype(jnp.float32).reshape(1, 1)
    n0nf = n0n.astype(jnp.float32).reshape(1, 1)

    eye4 = jnp.eye(4, dtype=jnp.float32)
    We1p = jnp.kron(eye4, We1)
    We2p = jnp.kron(eye4, We2)
    We3p = jnp.kron(eye4, We3)
    Wn1p = jnp.kron(eye4, Wn1)
    Wn2p = jnp.kron(eye4, Wn2)
    Wenc_e4 = jnp.kron(eye4, W_enc_e)
    Wenc_n4 = jnp.kron(eye4, W_enc_n)
    ffold = jnp.tile(jnp.eye(H, dtype=jnp.float32), (4, 1))

    h_g = jax.nn.relu(globals_ @ W_enc_g)
    h_n, tr, ts = _enc_nodes(nodes.reshape(N4, 512), Wenc_n4, We2p, We3p)
    h_e = _enc_edges(edges.reshape(E4, 64), Wenc_e4)

    def step(carry, _):
        h_e, h_n, tr, ts, h_g = carry
        gv = h_g @ We4
        gvt = jnp.tile(gv, (1, 4))
        gath = _sc_gather(tr, ts, rs)
        h_e, eagg = _edge_step(h_e, gath, n0ef, We1p, gvt, ffold)
        acc = _sc_scatter(h_e, r3)
        h_n, tr, ts, nagg = _node_step(h_n, acc[:N4], acc[NP4:NP4 + N4],
                                       n0nf, Wn1p, Wn2p, We2p, We3p, ffold)
        h_g = jax.nn.relu(
            jnp.concatenate([h_g, nagg, eagg], axis=-1) @ W_g)
        return (h_e, h_n, tr, ts, h_g), None

    (h_e, h_n, tr, ts, h_g), _ = lax.scan(
        step, (h_e, h_n, tr, ts, h_g), None, length=3)

    latent = h_g @ W_dec
    diff = latent[0] - latent[1]
    pred = jnp.tanh(jnp.dot(diff, w_rank))
    return pred.reshape(1)


# speedup vs baseline: 1.1647x; 1.1647x over previous
import functools

import jax
import jax.numpy as jnp
from jax import lax
from jax.experimental import pallas as pl
from jax.experimental.pallas import tpu as pltpu
from jax.experimental.pallas import tpu_sc as plsc

N = 10000
E = 320000
RK = 3
SK = 3
H = 32
NC = 2
NS = 16
NW = NC * NS
EPW = E // NW
C = 80
NCH = EPW // C
GCH = 25
NGR = NCH // GCH
NP = 10240
TPT = NP // NS
TABP = 2 * NP
TPW = TABP // NS

E4 = E // 4
C4 = C // 4
EB4 = 2000
NBE = E4 // EB4
N4 = N // 4
NP4 = NP // 4
NB4 = N4
NBN = N4 // NB4
THOP = N // C

_mesh = plsc.VectorSubcoreMesh(core_axis_name="c", subcore_axis_name="s")


@functools.partial(
    pl.kernel,
    out_type=jax.ShapeDtypeStruct((E4, 4 * H), jnp.float32),
    mesh=_mesh,
    scratch_types=[
        pltpu.VMEM((6, GCH, C), jnp.int32),
        pltpu.VMEM((2, 6, C, H), jnp.float32),
        pltpu.VMEM((2, C4, 4 * H), jnp.float32),
        pltpu.VMEM((C, H), jnp.float32),
        pltpu.VMEM_SHARED((TABP, H), jnp.float32),
        pltpu.SemaphoreType.DMA,
        pltpu.SemaphoreType.DMA,
    ],
    compiler_params=pltpu.CompilerParams(use_tc_tiling_on_sc=False),
)
def _sc_gather(trp_hbm, tsp_hbm, rs_hbm, out_hbm, idx6, rb, ob, stg, tab_sh,
               gsem, osem):
    c = lax.axis_index("c")
    s = lax.axis_index("s")
    wid = c * NS + s

    def stage_hop(it, carry):
        hh = s + it * NS

        @pl.when(hh < THOP)
        def _():
            for t, tp in enumerate([trp_hbm, tsp_hbm]):
                pltpu.sync_copy(tp.at[pl.ds(hh * C4, C4)], ob.at[0])

                def unpk(pr, carry2):
                    for slot in range(4):
                        for cc in range(2):
                            stg[4 * pr + slot, pl.ds(cc * 16, 16)] = (
                                ob[0, pr, pl.ds(slot * 32 + cc * 16, 16)])
                    return carry2

                lax.fori_loop(0, C4, unpk, 0)
                pltpu.sync_copy(stg, tab_sh.at[pl.ds(t * NP + hh * C, C)])
        return carry

    lax.fori_loop(0, (THOP + NS - 1) // NS, stage_hop, 0)
    plsc.subcore_barrier()

    def _fire(j, bank):
        for k in range(6):
            pltpu.async_copy(tab_sh.at[idx6.at[k, j]], rb.at[bank, k], gsem)

    def _waitg(j, bank):
        for k in range(6):
            pltpu.make_async_copy(
                tab_sh.at[idx6.at[k, j]], rb.at[bank, k], gsem).wait()

    def _drain_out(bank):
        pltpu.make_async_copy(
            ob.at[bank], out_hbm.at[pl.ds(0, C4)], osem).wait()

    def _compute(g, j, bank):
        @pl.when(g * GCH + j >= 2)
        def _():
            _drain_out(bank)

        def rowloop(pr, carry3):
            for slot in range(4):
                r = 4 * pr + slot
                for cc in range(2):
                    sl = pl.ds(cc * 16, 16)
                    osl = pl.ds(slot * 32 + cc * 16, 16)
                    ob[bank, pr, osl] = (
                        rb[bank, 0, r, sl] + rb[bank, 1, r, sl]
                        + rb[bank, 2, r, sl] + rb[bank, 3, r, sl]
                        + rb[bank, 4, r, sl] + rb[bank, 5, r, sl])
            return carry3

        lax.fori_loop(0, C4, rowloop, 0)
        pltpu.async_copy(
            ob.at[bank],
            out_hbm.at[pl.ds((wid * EPW + (g * GCH + j) * C) // 4, C4)],
            osem)

    def group(g, carry):
        for k in range(6):
            pltpu.sync_copy(rs_hbm.at[(k * NW + wid) * NGR + g], idx6.at[k])
        _fire(0, 0)

        def pair(jj, carry2):
            j0 = 2 * jj
            _fire(j0 + 1, 1)
            _waitg(j0, 0)
            _compute(g, j0, 0)
            _fire(j0 + 2, 0)
            _waitg(j0 + 1, 1)
            _compute(g, j0 + 1, 1)
            return carry2

        lax.fori_loop(0, (GCH - 1) // 2, pair, 0)
        _waitg(GCH - 1, 0)
        _compute(g, GCH - 1, 0)
        return carry

    lax.fori_loop(0, NGR, group, 0)
    _drain_out(0)
    _drain_out(1)


@functools.partial(
    pl.kernel,
    out_type=jax.ShapeDtypeStruct((NC * NP4, 4 * H), jnp.float32),
    mesh=_mesh,
    scratch_types=[
        pltpu.VMEM((3, GCH, C), jnp.int32),
        pltpu.VMEM((C, H), jnp.float32),
        pltpu.VMEM((C4, 4 * H), jnp.float32),
        pltpu.VMEM_SHARED((NP, H), jnp.float32),
    ],
    compiler_params=pltpu.CompilerParams(use_tc_tiling_on_sc=False),
)
def _sc_scatter(he_hbm, r_hbm, out_hbm, idx3, heb, heb2, acc_sh):
    c = lax.axis_index("c")
    s = lax.axis_index("s")
    wid = c * NS + s

    def zrow(i, carry):
        heb[i, pl.ds(0, 16)] = jnp.zeros((16,), jnp.float32)
        heb[i, pl.ds(16, 16)] = jnp.zeros((16,), jnp.float32)
        return carry

    lax.fori_loop(0, C, zrow, 0)

    def zhop(h, carry):
        pltpu.sync_copy(heb, acc_sh.at[pl.ds(s * TPT + h * C, C)])
        return carry

    lax.fori_loop(0, TPT // C, zhop, 0)
    plsc.subcore_barrier()

    def group(g, carry):
        for k in range(3):
            pltpu.sync_copy(r_hbm.at[(k * NW + wid) * NGR + g], idx3.at[k])

        def chunk(j, carry2):
            pltpu.sync_copy(
                he_hbm.at[pl.ds((wid * EPW + (g * GCH + j) * C) // 4, C4)],
                heb2)

            def unpack(pr, carry3):
                for slot in range(4):
                    for cc in range(2):
                        heb[4 * pr + slot, pl.ds(cc * 16, 16)] = (
                            heb2[pr, pl.ds(slot * 32 + cc * 16, 16)])
                return carry3

            lax.fori_loop(0, C4, unpack, 0)
            for k in range(3):
                pltpu.sync_copy(heb, acc_sh.at[idx3.at[k, j]], add=True)
            return carry2

        lax.fori_loop(0, GCH, chunk, 0)
        return carry

    lax.fori_loop(0, NGR, group, 0)
    plsc.subcore_barrier()

    def ohop(h, carry):
        row0 = s * TPT + h * C
        pltpu.sync_copy(acc_sh.at[pl.ds(row0, C)], heb)

        def pk(pr, carry2):
            for slot in range(4):
                for cc in range(2):
                    heb2[pr, pl.ds(slot * 32 + cc * 16, 16)] = (
                        heb[4 * pr + slot, pl.ds(cc * 16, 16)])
            return carry2

        lax.fori_loop(0, C4, pk, 0)
        pltpu.sync_copy(heb2, out_hbm.at[pl.ds((c * NP + row0) // 4, C4)])
        return carry

    lax.fori_loop(0, TPT // C, ohop, 0)


def _enc_nodes_body(x_ref, w_ref, w2_ref, w3_ref, hn_ref, tr_ref, ts_ref):
    h = jnp.maximum(
        jnp.dot(x_ref[...], w_ref[...], preferred_element_type=jnp.float32), 0.0)
    hn_ref[...] = h
    tr_ref[...] = jnp.dot(h, w2_ref[...], preferred_element_type=jnp.float32)
    ts_ref[...] = jnp.dot(h, w3_ref[...], preferred_element_type=jnp.float32)


def _enc_edges_body(x_ref, w_ref, he_ref):
    he_ref[...] = jnp.maximum(
        jnp.dot(x_ref[...], w_ref[...], preferred_element_type=jnp.float32), 0.0)


def _edge_body(he_ref, g_ref, n0_ref, w1_ref, gv_ref, f_ref, out_ref, agg_ref):
    b = pl.program_id(0)
    u = jnp.dot(he_ref[...], w1_ref[...],
                preferred_element_type=jnp.float32) + g_ref[...]
    ri = lax.broadcasted_iota(jnp.int32, (EB4, 128), 0) + b * EB4
    li = lax.broadcasted_iota(jnp.int32, (EB4, 128), 1)
    eidx = ri * 4 + (li >> 5)
    m0 = (eidx.astype(jnp.float32) < n0_ref[0, 0]).astype(jnp.float32)
    u = u + m0 * gv_ref[0:1, :] + (1.0 - m0) * gv_ref[1:2, :]
    hen = jnp.maximum(u, 0.0)
    out_ref[...] = hen
    s0 = jnp.sum(m0 * hen, axis=0, keepdims=True)
    st = jnp.sum(hen, axis=0, keepdims=True)
    p0 = jnp.dot(s0, f_ref[...], preferred_element_type=jnp.float32)
    pt = jnp.dot(st, f_ref[...], preferred_element_type=jnp.float32)
    part = jnp.concatenate([p0, pt - p0], axis=0)

    @pl.when(b == 0)
    def _():
        agg_ref[...] = part

    @pl.when(b != 0)
    def _():
        agg_ref[...] += part


def _node_body(hn_ref, a0_ref, a1_ref, n0_ref, w1_ref, w2_ref, we2_ref,
               we3_ref, f_ref, out_ref, tr_ref, ts_ref, agg_ref):
    b = pl.program_id(0)
    u = (jnp.dot(hn_ref[...], w1_ref[...], preferred_element_type=jnp.float32)
         + jnp.dot(a0_ref[...] + a1_ref[...], w2_ref[...],
                   preferred_element_type=jnp.float32))
    hnn = jnp.maximum(u, 0.0)
    out_ref[...] = hnn
    tr_ref[...] = jnp.dot(hnn, we2_ref[...], preferred_element_type=jnp.float32)
    ts_ref[...] = jnp.dot(hnn, we3_ref[...], preferred_element_type=jnp.float32)
    ri = lax.broadcasted_iota(jnp.int32, (NB4, 128), 0) + b * NB4
    li = lax.broadcasted_iota(jnp.int32, (NB4, 128), 1)
    nidx = ri * 4 + (li >> 5)
    m0 = (nidx.astype(jnp.float32) < n0_ref[0, 0]).astype(jnp.float32)
    s0 = jnp.sum(m0 * hnn, axis=0, keepdims=True)
    st = jnp.sum(hnn, axis=0, keepdims=True)
    p0 = jnp.dot(s0, f_ref[...], preferred_element_type=jnp.float32)
    pt = jnp.dot(st, f_ref[...], preferred_element_type=jnp.float32)
    part = jnp.concatenate([p0, pt - p0], axis=0)

    @pl.when(b == 0)
    def _():
        agg_ref[...] = part

    @pl.when(b != 0)
    def _():
        agg_ref[...] += part


def _wspec(shape):
    return pl.BlockSpec(shape, lambda i: (0,) * len(shape))


_enc_nodes = pl.pallas_call(
    _enc_nodes_body,
    grid=(NBN,),
    in_specs=[
        pl.BlockSpec((NB4, 512), lambda i: (i, 0)),
        _wspec((512, 4 * H)), _wspec((4 * H, 4 * H)), _wspec((4 * H, 4 * H)),
    ],
    out_specs=[
        pl.BlockSpec((NB4, 4 * H), lambda i: (i, 0)),
        pl.BlockSpec((NB4, 4 * H), lambda i: (i, 0)),
        pl.BlockSpec((NB4, 4 * H), lambda i: (i, 0)),
    ],
    out_shape=[jax.ShapeDtypeStruct((N4, 4 * H), jnp.float32)] * 3,
)

_enc_edges = pl.pallas_call(
    _enc_edges_body,
    grid=(NBE,),
    in_specs=[pl.BlockSpec((EB4, 64), lambda i: (i, 0)), _wspec((64, 4 * H))],
    out_specs=pl.BlockSpec((EB4, 4 * H), lambda i: (i, 0)),
    out_shape=jax.ShapeDtypeStruct((E4, 4 * H), jnp.float32),
)

_edge_step = pl.pallas_call(
    _edge_body,
    grid=(NBE,),
    in_specs=[
        pl.BlockSpec((EB4, 4 * H), lambda i: (i, 0)),
        pl.BlockSpec((EB4, 4 * H), lambda i: (i, 0)),
        _wspec((1, 1)),
        _wspec((4 * H, 4 * H)), _wspec((2, 4 * H)), _wspec((4 * H, H)),
    ],
    out_specs=[
        pl.BlockSpec((EB4, 4 * H), lambda i: (i, 0)),
        pl.BlockSpec((2, H), lambda i: (0, 0)),
    ],
    out_shape=[
        jax.ShapeDtypeStruct((E4, 4 * H), jnp.float32),
        jax.ShapeDtypeStruct((2, H), jnp.float32),
    ],
)

_node_step = pl.pallas_call(
    _node_body,
    grid=(NBN,),
    in_specs=[
        pl.BlockSpec((NB4, 4 * H), lambda i: (i, 0)),
        pl.BlockSpec((NB4, 4 * H), lambda i: (i, 0)),
        pl.BlockSpec((NB4, 4 * H), lambda i: (i, 0)),
        _wspec((1, 1)),
        _wspec((4 * H, 4 * H)), _wspec((4 * H, 4 * H)),
        _wspec((4 * H, 4 * H)), _wspec((4 * H, 4 * H)), _wspec((4 * H, H)),
    ],
    out_specs=[
        pl.BlockSpec((NB4, 4 * H), lambda i: (i, 0)),
        pl.BlockSpec((NB4, 4 * H), lambda i: (i, 0)),
        pl.BlockSpec((NB4, 4 * H), lambda i: (i, 0)),
        pl.BlockSpec((2, H), lambda i: (0, 0)),
    ],
    out_shape=[
        jax.ShapeDtypeStruct((N4, 4 * H), jnp.float32),
        jax.ShapeDtypeStruct((N4, 4 * H), jnp.float32),
        jax.ShapeDtypeStruct((N4, 4 * H), jnp.float32),
        jax.ShapeDtypeStruct((2, H), jnp.float32),
    ],
)


def kernel(nodes, edges, globals_, receivers, senders, node_graph_ids,
           edge_graph_ids, W_enc_n, W_enc_e, W_enc_g, W_e, W_n, W_g,
           W_dec, w_rank):
    recv = receivers.astype(jnp.int32)
    send = senders.astype(jnp.int32)
    rs = jnp.concatenate([recv.T, send.T + NP], axis=0)
    rs = rs.reshape(6 * NW * NGR, GCH, C)
    r3 = recv.T.reshape(3 * NW * NGR, GCH, C)

    We1, We2, We3, We4 = (W_e[0:H], W_e[H:2 * H], W_e[2 * H:3 * H], W_e[3 * H:])
    Wn1, Wn2 = W_n[0:H], W_n[H:]

    n0e = jnp.searchsorted(edge_graph_ids, 1)
    n0n = jnp.searchsorted(node_graph_ids, 1)
    n0ef = n0e.astype(jnp.float32).reshape(1, 1)
    n0nf = n0n.astype(jnp.float32).reshape(1, 1)

    eye4 = jnp.eye(4, dtype=jnp.float32)
    We1p = jnp.kron(eye4, We1)
    We2p = jnp.kron(eye4, We2)
    We3p = jnp.kron(eye4, We3)
    Wn1p = jnp.kron(eye4, Wn1)
    Wn2p = jnp.kron(eye4, Wn2)
    Wenc_e4 = jnp.kron(eye4, W_enc_e)
    Wenc_n4 = jnp.kron(eye4, W_enc_n)
    ffold = jnp.tile(jnp.eye(H, dtype=jnp.float32), (4, 1))

    h_g = jax.nn.relu(globals_ @ W_enc_g)
    h_n, tr, ts = _enc_nodes(nodes.reshape(N4, 512), Wenc_n4, We2p, We3p)
    h_e = _enc_edges(edges.reshape(E4, 64), Wenc_e4)

    def step(carry, _):
        h_e, h_n, tr, ts, h_g = carry
        gv = h_g @ We4
        gvt = jnp.tile(gv, (1, 4))
        gath = _sc_gather(tr, ts, rs)
        h_e, eagg = _edge_step(h_e, gath, n0ef, We1p, gvt, ffold)
        acc = _sc_scatter(h_e, r3)
        h_n, tr, ts, nagg = _node_step(h_n, acc[:N4], acc[NP4:NP4 + N4],
                                       n0nf, Wn1p, Wn2p, We2p, We3p, ffold)
        h_g = jax.nn.relu(
            jnp.concatenate([h_g, nagg, eagg], axis=-1) @ W_g)
        return (h_e, h_n, tr, ts, h_g), None

    (h_e, h_n, tr, ts, h_g), _ = lax.scan(
        step, (h_e, h_n, tr, ts, h_g), None, length=3)

    latent = h_g @ W_dec
    diff = latent[0] - latent[1]
    pred = jnp.tanh(jnp.dot(diff, w_rank))
    return pred.reshape(1)


# speedup vs baseline: 1.4582x; 1.2521x over previous
import functools

import jax
import jax.numpy as jnp
from jax import lax
from jax.experimental import pallas as pl
from jax.experimental.pallas import tpu as pltpu
from jax.experimental.pallas import tpu_sc as plsc

N = 10000
E = 320000
RK = 3
SK = 3
H = 32
NC = 2
NS = 16
NW = NC * NS
EPW = E // NW
C = 80
NCH = EPW // C
GCH = 25
NGR = NCH // GCH
NP = 10240
TPT = NP // NS
TABP = 2 * NP
TPW = TABP // NS

E4 = E // 4
C4 = C // 4
EB4 = 2000
NBE = E4 // EB4
N4 = N // 4
NP4 = NP // 4
NB4 = N4
NBN = N4 // NB4
THOP = N // C

_mesh = plsc.VectorSubcoreMesh(core_axis_name="c", subcore_axis_name="s")


@functools.partial(
    pl.kernel,
    out_type=jax.ShapeDtypeStruct((E4, 4 * H), jnp.float32),
    mesh=_mesh,
    scratch_types=[
        pltpu.VMEM((6, GCH, C), jnp.int32),
        pltpu.VMEM((2, 6, C, H), jnp.float32),
        pltpu.VMEM((2, C4, 4 * H), jnp.float32),
        pltpu.VMEM((C, H), jnp.float32),
        pltpu.VMEM_SHARED((TABP, H), jnp.float32),
        pltpu.SemaphoreType.DMA,
        pltpu.SemaphoreType.DMA,
    ],
    compiler_params=pltpu.CompilerParams(use_tc_tiling_on_sc=False),
)
def _sc_gather(trp_hbm, tsp_hbm, rs_hbm, out_hbm, idx6, rb, ob, stg, tab_sh,
               gsem, osem):
    c = lax.axis_index("c")
    s = lax.axis_index("s")
    wid = c * NS + s

    def stage_hop(it, carry):
        hh = s + it * NS

        @pl.when(hh < THOP)
        def _():
            for t, tp in enumerate([trp_hbm, tsp_hbm]):
                pltpu.sync_copy(tp.at[pl.ds(hh * C4, C4)], ob.at[0])

                def unpk(pr, carry2):
                    for slot in range(4):
                        for cc in range(2):
                            stg[4 * pr + slot, pl.ds(cc * 16, 16)] = (
                                ob[0, pr, pl.ds(slot * 32 + cc * 16, 16)])
                    return carry2

                lax.fori_loop(0, C4, unpk, 0)
                pltpu.sync_copy(stg, tab_sh.at[pl.ds(t * NP + hh * C, C)])
        return carry

    lax.fori_loop(0, (THOP + NS - 1) // NS, stage_hop, 0)
    plsc.subcore_barrier()

    def _fire(j, bank):
        for k in range(6):
            pltpu.async_copy(tab_sh.at[idx6.at[k, j]], rb.at[bank, k], gsem)

    def _waitg(j, bank):
        for k in range(6):
            pltpu.make_async_copy(
                tab_sh.at[idx6.at[k, j]], rb.at[bank, k], gsem).wait()

    def _drain_out(bank):
        pltpu.make_async_copy(
            ob.at[bank], out_hbm.at[pl.ds(0, C4)], osem).wait()

    def _compute(g, j, bank):
        @pl.when(g * GCH + j >= 2)
        def _():
            _drain_out(bank)

        def rowloop(pr, carry3):
            for slot in range(4):
                r = 4 * pr + slot
                for cc in range(2):
                    sl = pl.ds(cc * 16, 16)
                    osl = pl.ds(slot * 32 + cc * 16, 16)
                    ob[bank, pr, osl] = (
                        rb[bank, 0, r, sl] + rb[bank, 1, r, sl]
                        + rb[bank, 2, r, sl] + rb[bank, 3, r, sl]
                        + rb[bank, 4, r, sl] + rb[bank, 5, r, sl])
            return carry3

        lax.fori_loop(0, C4, rowloop, 0)
        pltpu.async_copy(
            ob.at[bank],
            out_hbm.at[pl.ds((wid * EPW + (g * GCH + j) * C) // 4, C4)],
            osem)

    def group(g, carry):
        for k in range(6):
            pltpu.sync_copy(rs_hbm.at[(k * NW + wid) * NGR + g], idx6.at[k])
        _fire(0, 0)

        def pair(jj, carry2):
            j0 = 2 * jj
            _fire(j0 + 1, 1)
            _waitg(j0, 0)
            _compute(g, j0, 0)
            _fire(j0 + 2, 0)
            _waitg(j0 + 1, 1)
            _compute(g, j0 + 1, 1)
            return carry2

        lax.fori_loop(0, (GCH - 1) // 2, pair, 0)
        _waitg(GCH - 1, 0)
        _compute(g, GCH - 1, 0)
        return carry

    lax.fori_loop(0, NGR, group, 0)
    _drain_out(0)
    _drain_out(1)


@functools.partial(
    pl.kernel,
    out_type=jax.ShapeDtypeStruct((NC * NP4, 4 * H), jnp.float32),
    mesh=_mesh,
    scratch_types=[
        pltpu.VMEM((3, GCH, C), jnp.int32),
        pltpu.VMEM((2, C, H), jnp.float32),
        pltpu.VMEM((2, C4, 4 * H), jnp.float32),
        pltpu.VMEM_SHARED((NP, H), jnp.float32),
        pltpu.SemaphoreType.DMA,
        pltpu.SemaphoreType.DMA,
    ],
    compiler_params=pltpu.CompilerParams(use_tc_tiling_on_sc=False),
)
def _sc_scatter(he_hbm, r_hbm, out_hbm, idx3, heb, heb2, acc_sh, lsem, ssem):
    c = lax.axis_index("c")
    s = lax.axis_index("s")
    wid = c * NS + s

    def zrow(i, carry):
        heb[0, i, pl.ds(0, 16)] = jnp.zeros((16,), jnp.float32)
        heb[0, i, pl.ds(16, 16)] = jnp.zeros((16,), jnp.float32)
        return carry

    lax.fori_loop(0, C, zrow, 0)

    def zhop(h, carry):
        pltpu.sync_copy(heb.at[0], acc_sh.at[pl.ds(s * TPT + h * C, C)])
        return carry

    lax.fori_loop(0, TPT // C, zhop, 0)
    plsc.subcore_barrier()

    def _fire_load(g, j, bank):
        pltpu.async_copy(
            he_hbm.at[pl.ds((wid * EPW + (g * GCH + j) * C) // 4, C4)],
            heb2.at[bank], lsem)

    def _wait_load(g, j, bank):
        pltpu.make_async_copy(
            he_hbm.at[pl.ds((wid * EPW + (g * GCH + j) * C) // 4, C4)],
            heb2.at[bank], lsem).wait()

    def _drain_adds(j, bank):
        for k in range(3):
            pltpu.make_async_copy(
                heb.at[bank], acc_sh.at[idx3.at[k, j]], ssem).wait()

    def _process(j, bank):
        @pl.when(j >= 2)
        def _():
            _drain_adds(j - 2, bank)

        def unpack(pr, carry3):
            for slot in range(4):
                for cc in range(2):
                    heb[bank, 4 * pr + slot, pl.ds(cc * 16, 16)] = (
                        heb2[bank, pr, pl.ds(slot * 32 + cc * 16, 16)])
            return carry3

        lax.fori_loop(0, C4, unpack, 0)
        for k in range(3):
            pltpu.async_copy(heb.at[bank], acc_sh.at[idx3.at[k, j]], ssem,
                             add=True)

    def group(g, carry):
        for k in range(3):
            pltpu.sync_copy(r_hbm.at[(k * NW + wid) * NGR + g], idx3.at[k])
        _fire_load(g, 0, 0)

        def pair(jj, carry2):
            j0 = 2 * jj
            _fire_load(g, j0 + 1, 1)
            _wait_load(g, j0, 0)
            _process(j0, 0)
            _fire_load(g, j0 + 2, 0)
            _wait_load(g, j0 + 1, 1)
            _process(j0 + 1, 1)
            return carry2

        lax.fori_loop(0, (GCH - 1) // 2, pair, 0)
        _wait_load(g, GCH - 1, 0)
        _process(GCH - 1, 0)
        _drain_adds(GCH - 2, 1)
        _drain_adds(GCH - 1, 0)
        return carry

    lax.fori_loop(0, NGR, group, 0)
    plsc.subcore_barrier()

    def ohop(h, carry):
        row0 = s * TPT + h * C
        pltpu.sync_copy(acc_sh.at[pl.ds(row0, C)], heb.at[0])

        def pk(pr, carry2):
            for slot in range(4):
                for cc in range(2):
                    heb2[0, pr, pl.ds(slot * 32 + cc * 16, 16)] = (
                        heb[0, 4 * pr + slot, pl.ds(cc * 16, 16)])
            return carry2

        lax.fori_loop(0, C4, pk, 0)
        pltpu.sync_copy(heb2.at[0],
                        out_hbm.at[pl.ds((c * NP + row0) // 4, C4)])
        return carry

    lax.fori_loop(0, TPT // C, ohop, 0)


def _enc_nodes_body(x_ref, w_ref, w2_ref, w3_ref, hn_ref, tr_ref, ts_ref):
    h = jnp.maximum(
        jnp.dot(x_ref[...], w_ref[...], preferred_element_type=jnp.float32), 0.0)
    hn_ref[...] = h
    tr_ref[...] = jnp.dot(h, w2_ref[...], preferred_element_type=jnp.float32)
    ts_ref[...] = jnp.dot(h, w3_ref[...], preferred_element_type=jnp.float32)


def _enc_edges_body(x_ref, w_ref, he_ref):
    he_ref[...] = jnp.maximum(
        jnp.dot(x_ref[...], w_ref[...], preferred_element_type=jnp.float32), 0.0)


def _edge_body(he_ref, g_ref, n0_ref, w1_ref, gv_ref, f_ref, out_ref, agg_ref):
    b = pl.program_id(0)
    u = jnp.dot(he_ref[...], w1_ref[...],
                preferred_element_type=jnp.float32) + g_ref[...]
    ri = lax.broadcasted_iota(jnp.int32, (EB4, 128), 0) + b * EB4
    li = lax.broadcasted_iota(jnp.int32, (EB4, 128), 1)
    eidx = ri * 4 + (li >> 5)
    m0 = (eidx.astype(jnp.float32) < n0_ref[0, 0]).astype(jnp.float32)
    u = u + m0 * gv_ref[0:1, :] + (1.0 - m0) * gv_ref[1:2, :]
    hen = jnp.maximum(u, 0.0)
    out_ref[...] = hen
    s0 = jnp.sum(m0 * hen, axis=0, keepdims=True)
    st = jnp.sum(hen, axis=0, keepdims=True)
    p0 = jnp.dot(s0, f_ref[...], preferred_element_type=jnp.float32)
    pt = jnp.dot(st, f_ref[...], preferred_element_type=jnp.float32)
    part = jnp.concatenate([p0, pt - p0], axis=0)

    @pl.when(b == 0)
    def _():
        agg_ref[...] = part

    @pl.when(b != 0)
    def _():
        agg_ref[...] += part


def _node_body(hn_ref, a0_ref, a1_ref, n0_ref, w1_ref, w2_ref, we2_ref,
               we3_ref, f_ref, out_ref, tr_ref, ts_ref, agg_ref):
    b = pl.program_id(0)
    u = (jnp.dot(hn_ref[...], w1_ref[...], preferred_element_type=jnp.float32)
         + jnp.dot(a0_ref[...] + a1_ref[...], w2_ref[...],
                   preferred_element_type=jnp.float32))
    hnn = jnp.maximum(u, 0.0)
    out_ref[...] = hnn
    tr_ref[...] = jnp.dot(hnn, we2_ref[...], preferred_element_type=jnp.float32)
    ts_ref[...] = jnp.dot(hnn, we3_ref[...], preferred_element_type=jnp.float32)
    ri = lax.broadcasted_iota(jnp.int32, (NB4, 128), 0) + b * NB4
    li = lax.broadcasted_iota(jnp.int32, (NB4, 128), 1)
    nidx = ri * 4 + (li >> 5)
    m0 = (nidx.astype(jnp.float32) < n0_ref[0, 0]).astype(jnp.float32)
    s0 = jnp.sum(m0 * hnn, axis=0, keepdims=True)
    st = jnp.sum(hnn, axis=0, keepdims=True)
    p0 = jnp.dot(s0, f_ref[...], preferred_element_type=jnp.float32)
    pt = jnp.dot(st, f_ref[...], preferred_element_type=jnp.float32)
    part = jnp.concatenate([p0, pt - p0], axis=0)

    @pl.when(b == 0)
    def _():
        agg_ref[...] = part

    @pl.when(b != 0)
    def _():
        agg_ref[...] += part


def _wspec(shape):
    return pl.BlockSpec(shape, lambda i: (0,) * len(shape))


_enc_nodes = pl.pallas_call(
    _enc_nodes_body,
    grid=(NBN,),
    in_specs=[
        pl.BlockSpec((NB4, 512), lambda i: (i, 0)),
        _wspec((512, 4 * H)), _wspec((4 * H, 4 * H)), _wspec((4 * H, 4 * H)),
    ],
    out_specs=[
        pl.BlockSpec((NB4, 4 * H), lambda i: (i, 0)),
        pl.BlockSpec((NB4, 4 * H), lambda i: (i, 0)),
        pl.BlockSpec((NB4, 4 * H), lambda i: (i, 0)),
    ],
    out_shape=[jax.ShapeDtypeStruct((N4, 4 * H), jnp.float32)] * 3,
)

_enc_edges = pl.pallas_call(
    _enc_edges_body,
    grid=(NBE,),
    in_specs=[pl.BlockSpec((EB4, 64), lambda i: (i, 0)), _wspec((64, 4 * H))],
    out_specs=pl.BlockSpec((EB4, 4 * H), lambda i: (i, 0)),
    out_shape=jax.ShapeDtypeStruct((E4, 4 * H), jnp.float32),
)

_edge_step = pl.pallas_call(
    _edge_body,
    grid=(NBE,),
    in_specs=[
        pl.BlockSpec((EB4, 4 * H), lambda i: (i, 0)),
        pl.BlockSpec((EB4, 4 * H), lambda i: (i, 0)),
        _wspec((1, 1)),
        _wspec((4 * H, 4 * H)), _wspec((2, 4 * H)), _wspec((4 * H, H)),
    ],
    out_specs=[
        pl.BlockSpec((EB4, 4 * H), lambda i: (i, 0)),
        pl.BlockSpec((2, H), lambda i: (0, 0)),
    ],
    out_shape=[
        jax.ShapeDtypeStruct((E4, 4 * H), jnp.float32),
        jax.ShapeDtypeStruct((2, H), jnp.float32),
    ],
)

_node_step = pl.pallas_call(
    _node_body,
    grid=(NBN,),
    in_specs=[
        pl.BlockSpec((NB4, 4 * H), lambda i: (i, 0)),
        pl.BlockSpec((NB4, 4 * H), lambda i: (i, 0)),
        pl.BlockSpec((NB4, 4 * H), lambda i: (i, 0)),
        _wspec((1, 1)),
        _wspec((4 * H, 4 * H)), _wspec((4 * H, 4 * H)),
        _wspec((4 * H, 4 * H)), _wspec((4 * H, 4 * H)), _wspec((4 * H, H)),
    ],
    out_specs=[
        pl.BlockSpec((NB4, 4 * H), lambda i: (i, 0)),
        pl.BlockSpec((NB4, 4 * H), lambda i: (i, 0)),
        pl.BlockSpec((NB4, 4 * H), lambda i: (i, 0)),
        pl.BlockSpec((2, H), lambda i: (0, 0)),
    ],
    out_shape=[
        jax.ShapeDtypeStruct((N4, 4 * H), jnp.float32),
        jax.ShapeDtypeStruct((N4, 4 * H), jnp.float32),
        jax.ShapeDtypeStruct((N4, 4 * H), jnp.float32),
        jax.ShapeDtypeStruct((2, H), jnp.float32),
    ],
)


def kernel(nodes, edges, globals_, receivers, senders, node_graph_ids,
           edge_graph_ids, W_enc_n, W_enc_e, W_enc_g, W_e, W_n, W_g,
           W_dec, w_rank):
    recv = receivers.astype(jnp.int32)
    send = senders.astype(jnp.int32)
    rs = jnp.concatenate([recv.T, send.T + NP], axis=0)
    rs = rs.reshape(6 * NW * NGR, GCH, C)
    r3 = recv.T.reshape(3 * NW * NGR, GCH, C)

    We1, We2, We3, We4 = (W_e[0:H], W_e[H:2 * H], W_e[2 * H:3 * H], W_e[3 * H:])
    Wn1, Wn2 = W_n[0:H], W_n[H:]

    n0e = jnp.searchsorted(edge_graph_ids, 1)
    n0n = jnp.searchsorted(node_graph_ids, 1)
    n0ef = n0e.astype(jnp.float32).reshape(1, 1)
    n0nf = n0n.astype(jnp.float32).reshape(1, 1)

    eye4 = jnp.eye(4, dtype=jnp.float32)
    We1p = jnp.kron(eye4, We1)
    We2p = jnp.kron(eye4, We2)
    We3p = jnp.kron(eye4, We3)
    Wn1p = jnp.kron(eye4, Wn1)
    Wn2p = jnp.kron(eye4, Wn2)
    Wenc_e4 = jnp.kron(eye4, W_enc_e)
    Wenc_n4 = jnp.kron(eye4, W_enc_n)
    ffold = jnp.tile(jnp.eye(H, dtype=jnp.float32), (4, 1))

    h_g = jax.nn.relu(globals_ @ W_enc_g)
    h_n, tr, ts = _enc_nodes(nodes.reshape(N4, 512), Wenc_n4, We2p, We3p)
    h_e = _enc_edges(edges.reshape(E4, 64), Wenc_e4)

    def step(carry, _):
        h_e, h_n, tr, ts, h_g = carry
        gv = h_g @ We4
        gvt = jnp.tile(gv, (1, 4))
        gath = _sc_gather(tr, ts, rs)
        h_e, eagg = _edge_step(h_e, gath, n0ef, We1p, gvt, ffold)
        acc = _sc_scatter(h_e, r3)
        h_n, tr, ts, nagg = _node_step(h_n, acc[:N4], acc[NP4:NP4 + N4],
                                       n0nf, Wn1p, Wn2p, We2p, We3p, ffold)
        h_g = jax.nn.relu(
            jnp.concatenate([h_g, nagg, eagg], axis=-1) @ W_g)
        return (h_e, h_n, tr, ts, h_g), None

    (h_e, h_n, tr, ts, h_g), _ = lax.scan(
        step, (h_e, h_n, tr, ts, h_g), None, length=3)

    latent = h_g @ W_dec
    diff = latent[0] - latent[1]
    pred = jnp.tanh(jnp.dot(diff, w_rank))
    return pred.reshape(1)
